# baseline (speedup 1.0000x reference)
import functools

import jax
import jax.numpy as jnp
from jax import lax
from jax.experimental import pallas as pl
from jax.experimental.pallas import tpu as pltpu
from jax.experimental.pallas import tpu_sc as plsc

_N = 10000
_E = 320000
_D = 128
_R = 8

_NC = 2
_NS = 16
_NW = _NC * _NS
_EPW = _E // _NW
_CH = 128
_NCHUNK = 80
_EPAD = _NW * _NCHUNK * _CH
_NPAD = 10112
_RPT = _NPAD // _NS

_BLK = 400
_GRID = _N // _BLK


def _l0_body(x_ref, w_ref, wself_ref, xw_ref, self_ref):
    r = pl.program_id(1)
    x = x_ref[...]
    xw_ref[0] = jnp.dot(x, w_ref[0], preferred_element_type=jnp.float32)

    @pl.when(r == 0)
    def _():
        self_ref[...] = jnp.dot(x, wself_ref[...],
                                preferred_element_type=jnp.float32)


def _flat_body(src_ref, typ_ref, dst_ref, flat_ref, dpad_ref):
    npad_rows = _EPAD // 128 - _E // 128
    flat_ref[...] = jnp.concatenate(
        [typ_ref[...] * _N + src_ref[...],
         jnp.zeros((npad_rows, 128), jnp.int32)], axis=0)
    dpad_ref[...] = jnp.concatenate(
        [dst_ref[...], jnp.full((npad_rows, 128), _N, jnp.int32)], axis=0)


def _tc_flat_idx(src2d, typ2d, dst2d):
    return pl.pallas_call(
        _flat_body,
        out_shape=[
            jax.ShapeDtypeStruct((_EPAD // 128, 128), jnp.int32),
            jax.ShapeDtypeStruct((_EPAD // 128, 128), jnp.int32),
        ],
    )(src2d, typ2d, dst2d)


def _mid_body(x_ref, self0_ref, parts_ref, b0_ref, w_ref, wself_ref,
              xw_ref, self_ref, h_scr):
    r = pl.program_id(1)

    @pl.when(r == 0)
    def _():
        h = x_ref[...] + self0_ref[...] + parts_ref[0] + parts_ref[1]
        h = jnp.maximum(h, 0.0) + b0_ref[...]
        h_scr[...] = h
        self_ref[...] = jnp.dot(h, wself_ref[...],
                                preferred_element_type=jnp.float32)

    xw_ref[0] = jnp.dot(h_scr[...], w_ref[0],
                        preferred_element_type=jnp.float32)


def _fin_body(self1_ref, parts_ref, b1_ref, o_ref):
    o_ref[...] = self1_ref[...] + parts_ref[0] + parts_ref[1] + b1_ref[...]


def _tc_layer0(x, W, wself):
    return pl.pallas_call(
        _l0_body,
        grid=(_GRID, _R),
        in_specs=[
            pl.BlockSpec((_BLK, _D), lambda i, r: (i, 0)),
            pl.BlockSpec((1, _D, _D), lambda i, r: (r, 0, 0)),
            pl.BlockSpec((_D, _D), lambda i, r: (0, 0)),
        ],
        out_specs=[
            pl.BlockSpec((1, _BLK, _D), lambda i, r: (r, i, 0)),
            pl.BlockSpec((_BLK, _D), lambda i, r: (i, 0)),
        ],
        out_shape=[
            jax.ShapeDtypeStruct((_R, _N, _D), jnp.float32),
            jax.ShapeDtypeStruct((_N, _D), jnp.float32),
        ],
    )(x, W, wself)


def _tc_mid(x, self0, parts, b0row, W, wself):
    return pl.pallas_call(
        _mid_body,
        grid=(_GRID, _R),
        in_specs=[
            pl.BlockSpec((_BLK, _D), lambda i, r: (i, 0)),
            pl.BlockSpec((_BLK, _D), lambda i, r: (i, 0)),
            pl.BlockSpec((_NC, _BLK, _D), lambda i, r: (0, i, 0)),
            pl.BlockSpec((1, _D), lambda i, r: (0, 0)),
            pl.BlockSpec((1, _D, _D), lambda i, r: (r, 0, 0)),
            pl.BlockSpec((_D, _D), lambda i, r: (0, 0)),
        ],
        out_specs=[
            pl.BlockSpec((1, _BLK, _D), lambda i, r: (r, i, 0)),
            pl.BlockSpec((_BLK, _D), lambda i, r: (i, 0)),
        ],
        out_shape=[
            jax.ShapeDtypeStruct((_R, _N, _D), jnp.float32),
            jax.ShapeDtypeStruct((_N, _D), jnp.float32),
        ],
        scratch_shapes=[pltpu.VMEM((_BLK, _D), jnp.float32)],
    )(x, self0, parts, b0row, W, wself)


def _tc_final(self1, parts, b1row):
    return pl.pallas_call(
        _fin_body,
        grid=(_GRID,),
        in_specs=[
            pl.BlockSpec((_BLK, _D), lambda i: (i, 0)),
            pl.BlockSpec((_NC, _BLK, _D), lambda i: (0, i, 0)),
            pl.BlockSpec((1, _D), lambda i: (0, 0)),
        ],
        out_specs=pl.BlockSpec((_BLK, _D), lambda i: (i, 0)),
        out_shape=jax.ShapeDtypeStruct((_N, _D), jnp.float32),
    )(self1, parts, b1row)


def _make_sc_kernel():
    mesh = plsc.VectorSubcoreMesh(core_axis_name="c", subcore_axis_name="s")

    def body(table, gidx, didx, zinit, out, idx_v, rows, agg_s, sem0, sem1):
        c = lax.axis_index("c")
        s = lax.axis_index("s")
        wid = s * _NC + c
        rows0 = rows.at[0]
        rows1 = rows.at[1]
        pltpu.sync_copy(zinit.at[pl.ds(s * _RPT, _RPT)],
                        agg_s.at[pl.ds(s * _RPT, _RPT)])
        plsc.subcore_barrier()

        pltpu.sync_copy(gidx.at[wid, 0], idx_v.at[0, 0])
        pltpu.sync_copy(didx.at[wid, 0], idx_v.at[0, 1])
        pltpu.async_copy(table.at[idx_v.at[0, 0]], rows0, sem0)
        pltpu.sync_copy(gidx.at[wid, 1], idx_v.at[1, 0])
        pltpu.sync_copy(didx.at[wid, 1], idx_v.at[1, 1])

        @pl.loop(0, _NCHUNK, step=2)
        def _chunk(j):
            pltpu.async_copy(table.at[idx_v.at[1, 0]], rows1, sem1)
            pltpu.make_async_copy(table.at[idx_v.at[0, 0]], rows0,
                                  sem0).wait()
            pltpu.sync_copy(rows0, agg_s.at[idx_v.at[0, 1]], add=True)

            @pl.when(j + 2 < _NCHUNK)
            def _next0():
                pltpu.sync_copy(gidx.at[wid, j + 2], idx_v.at[0, 0])
                pltpu.sync_copy(didx.at[wid, j + 2], idx_v.at[0, 1])
                pltpu.async_copy(table.at[idx_v.at[0, 0]], rows0, sem0)

            pltpu.make_async_copy(table.at[idx_v.at[1, 0]], rows1,
                                  sem1).wait()
            pltpu.sync_copy(rows1, agg_s.at[idx_v.at[1, 1]], add=True)

            @pl.when(j + 3 < _NCHUNK)
            def _next1():
                pltpu.sync_copy(gidx.at[wid, j + 3], idx_v.at[1, 0])
                pltpu.sync_copy(didx.at[wid, j + 3], idx_v.at[1, 1])

        plsc.subcore_barrier()
        pltpu.sync_copy(agg_s.at[pl.ds(s * _RPT, _RPT)],
                        out.at[c, pl.ds(s * _RPT, _RPT)])

    return pl.kernel(
        body,
        out_type=jax.ShapeDtypeStruct((_NC, _NPAD, _D), jnp.float32),
        mesh=mesh,
        scratch_types=[
            pltpu.VMEM((2, 2, _CH), jnp.int32),
            pltpu.VMEM((2, _CH, _D), jnp.float32),
            pltpu.VMEM_SHARED((_NPAD, _D), jnp.float32),
            pltpu.SemaphoreType.DMA,
            pltpu.SemaphoreType.DMA,
        ],
    )


@functools.cache
def _sc_kernel_cached():
    return _make_sc_kernel()


def _sc_gather_scatter(table, gidx, didx, zinit):
    return _sc_kernel_cached()(table, gidx, didx, zinit)


def kernel(x, edge_index, edge_type, W0, self_w0, b0, W1, self_w1, b1):
    src2d = edge_index[0].reshape(_E // 128, 128)
    typ2d = edge_type.reshape(_E // 128, 128)
    b0row = b0.reshape(1, _D)
    b1row = b1.reshape(1, _D)
    zinit = jnp.zeros((_NPAD, _D), jnp.float32)

    xw0, self0 = _tc_layer0(x, W0, self_w0)
    flat2d, dpad2d = _tc_flat_idx(src2d, typ2d,
                                  edge_index[1].reshape(_E // 128, 128))
    gidx = flat2d.reshape(_NW, _NCHUNK, _CH)
    didx = dpad2d.reshape(_NW, _NCHUNK, _CH)

    parts0 = _sc_gather_scatter(xw0.reshape(_R * _N, _D), gidx, didx, zinit)

    xw1, self1 = _tc_mid(x, self0, parts0, b0row, W1, self_w1)
    parts1 = _sc_gather_scatter(xw1.reshape(_R * _N, _D), gidx, didx, zinit)

    return _tc_final(self1, parts1, b1row)


# speedup vs baseline: 1.1697x; 1.1697x over previous
import functools

import jax
import jax.numpy as jnp
from jax import lax
from jax.experimental import pallas as pl
from jax.experimental.pallas import tpu as pltpu
from jax.experimental.pallas import tpu_sc as plsc

_N = 10000
_E = 320000
_D = 128
_R = 8

_NC = 2
_NS = 16
_NW = _NC * _NS
_EPW = _E // _NW
_CH = 128
_NCHUNK = 80
_EPAD = _NW * _NCHUNK * _CH
_NPAD = 10112
_RPT = _NPAD // _NS

_BLK = 400
_GRID = _N // _BLK


def _l0_body(x_ref, w_ref, wself_ref, xw_ref, self_ref):
    r = pl.program_id(1)
    x = x_ref[...]
    xw_ref[0] = jnp.dot(x, w_ref[0], preferred_element_type=jnp.float32)

    @pl.when(r == 0)
    def _():
        self_ref[...] = jnp.dot(x, wself_ref[...],
                                preferred_element_type=jnp.float32)


def _flat_body(src_ref, typ_ref, dst_ref, packed_ref):
    npad_rows = _EPAD // 128 - _E // 128
    flat = typ_ref[...] * _N + src_ref[...]
    packed_ref[...] = jnp.concatenate(
        [flat * 16384 + dst_ref[...],
         jnp.full((npad_rows, 128), _N, jnp.int32)], axis=0)


def _tc_flat_idx(src2d, typ2d, dst2d):
    return pl.pallas_call(
        _flat_body,
        out_shape=jax.ShapeDtypeStruct((_EPAD // 128, 128), jnp.int32),
    )(src2d, typ2d, dst2d)


def _mid_body(x_ref, self0_ref, parts_ref, b0_ref, w_ref, wself_ref,
              xw_ref, self_ref, h_scr):
    r = pl.program_id(1)

    @pl.when(r == 0)
    def _():
        h = x_ref[...] + self0_ref[...] + parts_ref[0] + parts_ref[1]
        h = jnp.maximum(h, 0.0) + b0_ref[...]
        h_scr[...] = h
        self_ref[...] = jnp.dot(h, wself_ref[...],
                                preferred_element_type=jnp.float32)

    xw_ref[0] = jnp.dot(h_scr[...], w_ref[0],
                        preferred_element_type=jnp.float32)


def _fin_body(self1_ref, parts_ref, b1_ref, o_ref):
    o_ref[...] = self1_ref[...] + parts_ref[0] + parts_ref[1] + b1_ref[...]


def _tc_layer0(x, W, wself):
    return pl.pallas_call(
        _l0_body,
        grid=(_GRID, _R),
        in_specs=[
            pl.BlockSpec((_BLK, _D), lambda i, r: (i, 0)),
            pl.BlockSpec((1, _D, _D), lambda i, r: (r, 0, 0)),
            pl.BlockSpec((_D, _D), lambda i, r: (0, 0)),
        ],
        out_specs=[
            pl.BlockSpec((1, _BLK, _D), lambda i, r: (r, i, 0)),
            pl.BlockSpec((_BLK, _D), lambda i, r: (i, 0)),
        ],
        out_shape=[
            jax.ShapeDtypeStruct((_R, _N, _D), jnp.float32),
            jax.ShapeDtypeStruct((_N, _D), jnp.float32),
        ],
    )(x, W, wself)


def _tc_mid(x, self0, parts, b0row, W, wself):
    return pl.pallas_call(
        _mid_body,
        grid=(_GRID, _R),
        in_specs=[
            pl.BlockSpec((_BLK, _D), lambda i, r: (i, 0)),
            pl.BlockSpec((_BLK, _D), lambda i, r: (i, 0)),
            pl.BlockSpec((_NC, _BLK, _D), lambda i, r: (0, i, 0)),
            pl.BlockSpec((1, _D), lambda i, r: (0, 0)),
            pl.BlockSpec((1, _D, _D), lambda i, r: (r, 0, 0)),
            pl.BlockSpec((_D, _D), lambda i, r: (0, 0)),
        ],
        out_specs=[
            pl.BlockSpec((1, _BLK, _D), lambda i, r: (r, i, 0)),
            pl.BlockSpec((_BLK, _D), lambda i, r: (i, 0)),
        ],
        out_shape=[
            jax.ShapeDtypeStruct((_R, _N, _D), jnp.float32),
            jax.ShapeDtypeStruct((_N, _D), jnp.float32),
        ],
        scratch_shapes=[pltpu.VMEM((_BLK, _D), jnp.float32)],
    )(x, self0, parts, b0row, W, wself)


def _tc_final(self1, parts, b1row):
    return pl.pallas_call(
        _fin_body,
        grid=(_GRID,),
        in_specs=[
            pl.BlockSpec((_BLK, _D), lambda i: (i, 0)),
            pl.BlockSpec((_NC, _BLK, _D), lambda i: (0, i, 0)),
            pl.BlockSpec((1, _D), lambda i: (0, 0)),
        ],
        out_specs=pl.BlockSpec((_BLK, _D), lambda i: (i, 0)),
        out_shape=jax.ShapeDtypeStruct((_N, _D), jnp.float32),
    )(self1, parts, b1row)


def _make_sc_kernel():
    mesh = plsc.VectorSubcoreMesh(core_axis_name="c", subcore_axis_name="s")

    def body(table, packed, zinit, out, packed_v, idx_v, rows, agg_s,
             sem0, sem1):
        c = lax.axis_index("c")
        s = lax.axis_index("s")
        wid = s * _NC + c
        rows0 = rows.at[0]
        rows1 = rows.at[1]
        pltpu.sync_copy(packed.at[wid], packed_v)
        pltpu.sync_copy(zinit.at[pl.ds(s * _RPT, _RPT)],
                        agg_s.at[pl.ds(s * _RPT, _RPT)])

        def unpack(j, b):
            for k in range(_CH // 16):
                p = packed_v[j, pl.ds(k * 16, 16)]
                idx_v[b, 0, pl.ds(k * 16, 16)] = lax.shift_right_logical(
                    p, 14)
                idx_v[b, 1, pl.ds(k * 16, 16)] = lax.bitwise_and(p, 16383)

        plsc.subcore_barrier()

        unpack(0, 0)
        pltpu.async_copy(table.at[idx_v.at[0, 0]], rows0, sem0)
        unpack(1, 1)

        @pl.loop(0, _NCHUNK, step=2)
        def _chunk(j):
            pltpu.async_copy(table.at[idx_v.at[1, 0]], rows1, sem1)
            pltpu.make_async_copy(table.at[idx_v.at[0, 0]], rows0,
                                  sem0).wait()
            pltpu.sync_copy(rows0, agg_s.at[idx_v.at[0, 1]], add=True)

            @pl.when(j + 2 < _NCHUNK)
            def _next0():
                unpack(j + 2, 0)
                pltpu.async_copy(table.at[idx_v.at[0, 0]], rows0, sem0)

            pltpu.make_async_copy(table.at[idx_v.at[1, 0]], rows1,
                                  sem1).wait()
            pltpu.sync_copy(rows1, agg_s.at[idx_v.at[1, 1]], add=True)

            @pl.when(j + 3 < _NCHUNK)
            def _next1():
                unpack(j + 3, 1)

        plsc.subcore_barrier()
        pltpu.sync_copy(agg_s.at[pl.ds(s * _RPT, _RPT)],
                        out.at[c, pl.ds(s * _RPT, _RPT)])

    return pl.kernel(
        body,
        out_type=jax.ShapeDtypeStruct((_NC, _NPAD, _D), jnp.float32),
        mesh=mesh,
        scratch_types=[
            pltpu.VMEM((_NCHUNK, _CH), jnp.int32),
            pltpu.VMEM((2, 2, _CH), jnp.int32),
            pltpu.VMEM((2, _CH, _D), jnp.float32),
            pltpu.VMEM_SHARED((_NPAD, _D), jnp.float32),
            pltpu.SemaphoreType.DMA,
            pltpu.SemaphoreType.DMA,
        ],
    )


@functools.cache
def _sc_kernel_cached():
    return _make_sc_kernel()


def _sc_gather_scatter(table, packed, zinit):
    return _sc_kernel_cached()(table, packed, zinit)


def kernel(x, edge_index, edge_type, W0, self_w0, b0, W1, self_w1, b1):
    src2d = edge_index[0].reshape(_E // 128, 128)
    typ2d = edge_type.reshape(_E // 128, 128)
    b0row = b0.reshape(1, _D)
    b1row = b1.reshape(1, _D)
    zinit = jnp.zeros((_NPAD, _D), jnp.float32)

    xw0, self0 = _tc_layer0(x, W0, self_w0)
    packed2d = _tc_flat_idx(src2d, typ2d,
                            edge_index[1].reshape(_E // 128, 128))
    packed = packed2d.reshape(_NW, _NCHUNK, _CH)

    parts0 = _sc_gather_scatter(xw0.reshape(_R * _N, _D), packed, zinit)

    xw1, self1 = _tc_mid(x, self0, parts0, b0row, W1, self_w1)
    parts1 = _sc_gather_scatter(xw1.reshape(_R * _N, _D), packed, zinit)

    return _tc_final(self1, parts1, b1row)


# speedup vs baseline: 1.4159x; 1.2105x over previous
import functools

import jax
import jax.numpy as jnp
from jax import lax
from jax.experimental import pallas as pl
from jax.experimental.pallas import tpu as pltpu
from jax.experimental.pallas import tpu_sc as plsc

_N = 10000
_E = 320000
_D = 128
_R = 8

_NC = 2
_NS = 16
_NW = _NC * _NS
_EPW = _E // _NW
_CH = 128
_NCHUNK = 80
_EPAD = _NW * _NCHUNK * _CH
_NPAD = 10112
_RPT = _NPAD // _NS

_BLK = 400
_GRID = _N // _BLK


def _l0_body(x_ref, w_ref, wself_ref, xw_ref, self_ref):
    x = x_ref[...]
    for r in range(_R):
        xw_ref[:, r, :] = jnp.dot(x, w_ref[r],
                                  preferred_element_type=jnp.float32)
    self_ref[...] = jnp.dot(x, wself_ref[...],
                            preferred_element_type=jnp.float32)


def _flat_body(src_ref, typ_ref, dst_ref, packed_ref):
    npad_rows = _EPAD // 128 - _E // 128
    flat = src_ref[...] * _R + typ_ref[...]
    packed_ref[...] = jnp.concatenate(
        [flat * 16384 + dst_ref[...],
         jnp.full((npad_rows, 128), _N, jnp.int32)], axis=0)


def _tc_flat_idx(src2d, typ2d, dst2d):
    return pl.pallas_call(
        _flat_body,
        out_shape=jax.ShapeDtypeStruct((_EPAD // 128, 128), jnp.int32),
    )(src2d, typ2d, dst2d)


def _mid_body(x_ref, self0_ref, parts_ref, b0_ref, w_ref, wself_ref,
              xw_ref, self_ref):
    h = x_ref[...] + self0_ref[...] + parts_ref[0] + parts_ref[1]
    h = jnp.maximum(h, 0.0) + b0_ref[...]
    self_ref[...] = jnp.dot(h, wself_ref[...],
                            preferred_element_type=jnp.float32)
    for r in range(_R):
        xw_ref[:, r, :] = jnp.dot(h, w_ref[r],
                                  preferred_element_type=jnp.float32)


def _fin_body(self1_ref, parts_ref, b1_ref, o_ref):
    o_ref[...] = self1_ref[...] + parts_ref[0] + parts_ref[1] + b1_ref[...]


def _tc_layer0(x, W, wself):
    return pl.pallas_call(
        _l0_body,
        grid=(_GRID,),
        in_specs=[
            pl.BlockSpec((_BLK, _D), lambda i: (i, 0)),
            pl.BlockSpec((_R, _D, _D), lambda i: (0, 0, 0)),
            pl.BlockSpec((_D, _D), lambda i: (0, 0)),
        ],
        out_specs=[
            pl.BlockSpec((_BLK, _R, _D), lambda i: (i, 0, 0)),
            pl.BlockSpec((_BLK, _D), lambda i: (i, 0)),
        ],
        out_shape=[
            jax.ShapeDtypeStruct((_N, _R, _D), jnp.float32),
            jax.ShapeDtypeStruct((_N, _D), jnp.float32),
        ],
    )(x, W, wself)


def _tc_mid(x, self0, parts, b0row, W, wself):
    return pl.pallas_call(
        _mid_body,
        grid=(_GRID,),
        in_specs=[
            pl.BlockSpec((_BLK, _D), lambda i: (i, 0)),
            pl.BlockSpec((_BLK, _D), lambda i: (i, 0)),
            pl.BlockSpec((_NC, _BLK, _D), lambda i: (0, i, 0)),
            pl.BlockSpec((1, _D), lambda i: (0, 0)),
            pl.BlockSpec((_R, _D, _D), lambda i: (0, 0, 0)),
            pl.BlockSpec((_D, _D), lambda i: (0, 0)),
        ],
        out_specs=[
            pl.BlockSpec((_BLK, _R, _D), lambda i: (i, 0, 0)),
            pl.BlockSpec((_BLK, _D), lambda i: (i, 0)),
        ],
        out_shape=[
            jax.ShapeDtypeStruct((_N, _R, _D), jnp.float32),
            jax.ShapeDtypeStruct((_N, _D), jnp.float32),
        ],
    )(x, self0, parts, b0row, W, wself)


def _tc_final(self1, parts, b1row):
    return pl.pallas_call(
        _fin_body,
        grid=(_GRID,),
        in_specs=[
            pl.BlockSpec((_BLK, _D), lambda i: (i, 0)),
            pl.BlockSpec((_NC, _BLK, _D), lambda i: (0, i, 0)),
            pl.BlockSpec((1, _D), lambda i: (0, 0)),
        ],
        out_specs=pl.BlockSpec((_BLK, _D), lambda i: (i, 0)),
        out_shape=jax.ShapeDtypeStruct((_N, _D), jnp.float32),
    )(self1, parts, b1row)


def _make_sc_kernel():
    mesh = plsc.VectorSubcoreMesh(core_axis_name="c", subcore_axis_name="s")

    def body(table, packed, zinit, out, packed_v, idx_v, rows, agg_s,
             sem0, sem1):
        c = lax.axis_index("c")
        s = lax.axis_index("s")
        wid = s * _NC + c
        rows0 = rows.at[0]
        rows1 = rows.at[1]
        pltpu.sync_copy(packed.at[wid], packed_v)
        pltpu.sync_copy(zinit.at[pl.ds(s * _RPT, _RPT)],
                        agg_s.at[pl.ds(s * _RPT, _RPT)])

        def unpack(j, b):
            for k in range(_CH // 16):
                p = packed_v[j, pl.ds(k * 16, 16)]
                idx_v[b, 0, pl.ds(k * 16, 16)] = lax.shift_right_logical(
                    p, 14)
                idx_v[b, 1, pl.ds(k * 16, 16)] = lax.bitwise_and(p, 16383)

        plsc.subcore_barrier()

        unpack(0, 0)
        pltpu.async_copy(table.at[idx_v.at[0, 0]], rows0, sem0)
        unpack(1, 1)

        @pl.loop(0, _NCHUNK, step=2)
        def _chunk(j):
            pltpu.async_copy(table.at[idx_v.at[1, 0]], rows1, sem1)
            pltpu.make_async_copy(table.at[idx_v.at[0, 0]], rows0,
                                  sem0).wait()
            pltpu.sync_copy(rows0, agg_s.at[idx_v.at[0, 1]], add=True)

            @pl.when(j + 2 < _NCHUNK)
            def _next0():
                unpack(j + 2, 0)
                pltpu.async_copy(table.at[idx_v.at[0, 0]], rows0, sem0)

            pltpu.make_async_copy(table.at[idx_v.at[1, 0]], rows1,
                                  sem1).wait()
            pltpu.sync_copy(rows1, agg_s.at[idx_v.at[1, 1]], add=True)

            @pl.when(j + 3 < _NCHUNK)
            def _next1():
                unpack(j + 3, 1)

        plsc.subcore_barrier()
        pltpu.sync_copy(agg_s.at[pl.ds(s * _RPT, _RPT)],
                        out.at[c, pl.ds(s * _RPT, _RPT)])

    return pl.kernel(
        body,
        out_type=jax.ShapeDtypeStruct((_NC, _NPAD, _D), jnp.float32),
        mesh=mesh,
        scratch_types=[
            pltpu.VMEM((_NCHUNK, _CH), jnp.int32),
            pltpu.VMEM((2, 2, _CH), jnp.int32),
            pltpu.VMEM((2, _CH, _D), jnp.float32),
            pltpu.VMEM_SHARED((_NPAD, _D), jnp.float32),
            pltpu.SemaphoreType.DMA,
            pltpu.SemaphoreType.DMA,
        ],
    )


@functools.cache
def _sc_kernel_cached():
    return _make_sc_kernel()


def _sc_gather_scatter(table, packed, zinit):
    return _sc_kernel_cached()(table, packed, zinit)


def kernel(x, edge_index, edge_type, W0, self_w0, b0, W1, self_w1, b1):
    src2d = edge_index[0].reshape(_E // 128, 128)
    typ2d = edge_type.reshape(_E // 128, 128)
    b0row = b0.reshape(1, _D)
    b1row = b1.reshape(1, _D)
    zinit = jnp.zeros((_NPAD, _D), jnp.float32)

    xw0, self0 = _tc_layer0(x, W0, self_w0)
    packed2d = _tc_flat_idx(src2d, typ2d,
                            edge_index[1].reshape(_E // 128, 128))
    packed = packed2d.reshape(_NW, _NCHUNK, _CH)

    parts0 = _sc_gather_scatter(xw0.reshape(_N * _R, _D), packed, zinit)

    xw1, self1 = _tc_mid(x, self0, parts0, b0row, W1, self_w1)
    parts1 = _sc_gather_scatter(xw1.reshape(_N * _R, _D), packed, zinit)

    return _tc_final(self1, parts1, b1row)
